# Initial kernel scaffold; baseline (speedup 1.0000x reference)
import jax
import jax.numpy as jnp
from jax.experimental import pallas as pl


def kernel(x, edge_index, edge_attr, batch_idx, W_emb, b_emb, W0, b0, gamma0, beta0, mmean0, mvar0, W1, b1, gamma1, beta1, mmean1, mvar1, W2, b2, gamma2, beta2, mmean2, mvar2, W_post, b_post, W_out, b_out):
    raise NotImplementedError("write your pallas kernel here")


# speedup vs baseline: 3.1774x; 3.1774x over previous
import functools

import jax
import jax.numpy as jnp
from jax import lax
from jax.experimental import pallas as pl
from jax.experimental.pallas import tpu as pltpu
from jax.experimental.pallas import tpu_sc as plsc

N = 10000
D = 128
DE = 16
NG = 64
EPS = 1e-3

NP = 10240
NACC = 10112
NB = 256
GRID = NP // NB
NW = 32
CHUNK = 128
CPW = 80
EPW = CPW * CHUNK
EP = NW * EPW
RPS = NACC // 16
DUMMY = N
NBUF = 2

_MESH = plsc.VectorSubcoreMesh(core_axis_name="c", subcore_axis_name="s")
_HIGH = jax.lax.Precision.HIGHEST


def _q(a):
    v = jax.lax.bitcast_convert_type(a, jnp.uint32)
    r = (v + jnp.uint32(0x7FFF) + ((v >> 16) & jnp.uint32(1))) \
        & jnp.uint32(0xFFFF0000)
    return jax.lax.bitcast_convert_type(r, jnp.float32)


@functools.partial(
    pl.kernel,
    out_type=jax.ShapeDtypeStruct((2 * NP, D), jnp.float32),
    mesh=_MESH,
    scratch_types=[
        pltpu.VMEM((NBUF, 2, CHUNK), jnp.int32),
        pltpu.VMEM((NBUF, CHUNK, D), jnp.float32),
        pltpu.VMEM_SHARED((NACC, D), jnp.float32),
        pltpu.SemaphoreType.DMA,
        pltpu.SemaphoreType.DMA,
    ],
)
def _sc_neighbor_sum(h_hbm, idx_hbm, zeros_hbm, out_hbm,
                     idxv, rows, acc, isem, gsem):
    c = lax.axis_index("c")
    s = lax.axis_index("s")
    wid = s * 2 + c
    pltpu.sync_copy(zeros_hbm, acc.at[pl.ds(s * RPS, RPS)])

    def idx_load(j, b):
        return pltpu.async_copy(idx_hbm.at[wid].at[j], idxv.at[b], isem)

    def idx_wait(j, b):
        pltpu.make_async_copy(idx_hbm.at[wid].at[j], idxv.at[b], isem).wait()

    def gather(b):
        return pltpu.async_copy(h_hbm.at[idxv.at[b].at[0]], rows.at[b], gsem)

    def gather_wait(b):
        pltpu.make_async_copy(h_hbm.at[idxv.at[b].at[0]], rows.at[b],
                              gsem).wait()

    for b in range(NBUF):
        idx_load(b, b)
    plsc.subcore_barrier()
    for b in range(NBUF - 1):
        idx_wait(b, b)
        gather(b)

    def body(i, _):
        for b in range(NBUF):
            j = i * NBUF + b
            b2 = (b + NBUF - 1) % NBUF

            @pl.when(j + NBUF - 1 < CPW)
            def _():
                idx_wait(j + NBUF - 1, b2)
                gather(b2)

            gather_wait(b)
            pltpu.sync_copy(rows.at[b], acc.at[idxv.at[b].at[1]], add=True)

            @pl.when(j + NBUF < CPW)
            def _():
                idx_load(j + NBUF, b)
        return ()

    lax.fori_loop(0, CPW // NBUF, body, (), unroll=False)
    plsc.subcore_barrier()
    pltpu.sync_copy(acc.at[pl.ds(s * RPS, RPS)],
                    out_hbm.at[pl.ds(c * NP + s * RPS, RPS)])


@functools.partial(
    pl.kernel,
    out_type=jax.ShapeDtypeStruct((2 * NP, D), jnp.float32),
    mesh=_MESH,
    scratch_types=[
        pltpu.VMEM((NBUF, 2, CHUNK), jnp.int32),
        pltpu.VMEM((NBUF, CHUNK, D), jnp.float32),
        pltpu.VMEM_SHARED((NACC, D), jnp.float32),
        pltpu.SemaphoreType.DMA,
        pltpu.SemaphoreType.DMA,
    ],
)
def _sc_edge_stats(ea_hbm, idx_hbm, zeros_hbm, out_hbm,
                   idxv, eav, acc, isem, gsem):
    c = lax.axis_index("c")
    s = lax.axis_index("s")
    wid = s * 2 + c
    pltpu.sync_copy(zeros_hbm, acc.at[pl.ds(s * RPS, RPS)])

    def load(j, b):
        pltpu.async_copy(idx_hbm.at[wid].at[j], idxv.at[b], isem)
        pltpu.async_copy(ea_hbm.at[wid].at[j], eav.at[b], gsem)

    def load_wait(j, b):
        pltpu.make_async_copy(idx_hbm.at[wid].at[j], idxv.at[b],
                              isem).wait()
        pltpu.make_async_copy(ea_hbm.at[wid].at[j], eav.at[b], gsem).wait()

    for b in range(NBUF):
        load(b, b)
    plsc.subcore_barrier()

    def body(i, _):
        for b in range(NBUF):
            j = i * NBUF + b
            load_wait(j, b)
            pltpu.sync_copy(eav.at[b], acc.at[idxv.at[b].at[1]], add=True)

            @pl.when(j + NBUF < CPW)
            def _():
                load(j + NBUF, b)
        return ()

    lax.fori_loop(0, CPW // NBUF, body, (), unroll=False)
    plsc.subcore_barrier()
    pltpu.sync_copy(acc.at[pl.ds(s * RPS, RPS)],
                    out_hbm.at[pl.ds(c * NP + s * RPS, RPS)])


def _embed_body(x_ref, w_ref, b_ref, o_ref, oq_ref):
    h = jax.nn.relu(
        jnp.dot(x_ref[...], w_ref[...], precision=_HIGH) + b_ref[...])
    o_ref[...] = h
    oq_ref[...] = _q(h)


def _embed(x, w, b):
    blk = pl.BlockSpec((NB, D), lambda i: (i, 0))
    return pl.pallas_call(
        _embed_body,
        grid=(GRID,),
        in_specs=[
            blk,
            pl.BlockSpec((D, D), lambda i: (0, 0)),
            pl.BlockSpec((1, D), lambda i: (0, 0)),
        ],
        out_specs=(blk, blk),
        out_shape=(jax.ShapeDtypeStruct((NP, D), jnp.float32),
                   jax.ShapeDtypeStruct((NP, D), jnp.float32)),
    )(x, w, b)


def _layer_body(s0_ref, s1_ref, st0_ref, st1_ref, h_ref,
                wt_ref, wb_ref, sc_ref, sh_ref, o_ref, oq_ref):
    i = pl.program_id(0)
    rid = i * NB + lax.broadcasted_iota(jnp.int32, (NB, 1), 0)
    valid = rid < NACC
    ssum = jnp.where(valid, s0_ref[...] + s1_ref[...], 0.0)
    st = jnp.where(valid, st0_ref[...] + st1_ref[...], 0.0)
    easum = st[:, :DE]
    deg = jnp.maximum(st[:, DE:DE + 1], 1.0)
    t = (jnp.dot(ssum, wt_ref[...], precision=_HIGH)
         + jnp.dot(easum, wb_ref[...], precision=_HIGH))
    agg = t * (sc_ref[...] / deg) + sh_ref[...]
    h = h_ref[...] + jax.nn.relu(agg)
    o_ref[...] = h
    oq_ref[...] = _q(h)


def _layer_update(s2, st2, h, w_top, w_bot, scale, shift):
    half = pl.BlockSpec((NB, D), lambda i: (i, 0))
    half_hi = pl.BlockSpec((NB, D), lambda i: (i + GRID, 0))
    return pl.pallas_call(
        _layer_body,
        grid=(GRID,),
        in_specs=[
            half, half_hi, half, half_hi,
            pl.BlockSpec((NB, D), lambda i: (i, 0)),
            pl.BlockSpec((D, D), lambda i: (0, 0)),
            pl.BlockSpec((DE, D), lambda i: (0, 0)),
            pl.BlockSpec((1, D), lambda i: (0, 0)),
            pl.BlockSpec((1, D), lambda i: (0, 0)),
        ],
        out_specs=(pl.BlockSpec((NB, D), lambda i: (i, 0)),
                   pl.BlockSpec((NB, D), lambda i: (i, 0))),
        out_shape=(jax.ShapeDtypeStruct((NP, D), jnp.float32),
                   jax.ShapeDtypeStruct((NP, D), jnp.float32)),
    )(s2, s2, st2, st2, h, w_top, w_bot, scale, shift)


def _pool_body(h_ref, bi_ref, wp_ref, bp_ref, wo_ref, bo_ref, o_ref,
               pool_ref, cnt_ref):
    i = pl.program_id(0)

    @pl.when(i == 0)
    def _():
        pool_ref[...] = jnp.zeros_like(pool_ref)
        cnt_ref[...] = jnp.zeros_like(cnt_ref)

    onehot = (lax.broadcasted_iota(jnp.int32, (NG, NB), 0)
              == bi_ref[0]).astype(jnp.float32)
    pool_ref[...] += jnp.dot(onehot, h_ref[...], precision=_HIGH)
    cnt_ref[...] += jnp.dot(onehot, jnp.ones((NB, D), jnp.float32),
                            precision=_HIGH)

    @pl.when(i == GRID - 1)
    def _():
        pooled = _q(pool_ref[...] / cnt_ref[...])
        hp = _q(jax.nn.relu(
            jnp.dot(pooled, wp_ref[...], precision=_HIGH) + bp_ref[...]))
        o_ref[...] = jnp.dot(hp, wo_ref[...], precision=_HIGH) + bo_ref[...]


def _pool_head(h, bi3, w_post, b_post, w_out, b_out):
    return pl.pallas_call(
        _pool_body,
        grid=(GRID,),
        in_specs=[
            pl.BlockSpec((NB, D), lambda i: (i, 0)),
            pl.BlockSpec((1, 1, NB), lambda i: (i, 0, 0)),
            pl.BlockSpec((D, D), lambda i: (0, 0)),
            pl.BlockSpec((1, D), lambda i: (0, 0)),
            pl.BlockSpec((D, 1), lambda i: (0, 0)),
            pl.BlockSpec((1, 1), lambda i: (0, 0)),
        ],
        out_specs=pl.BlockSpec((NG, 1), lambda i: (0, 0)),
        out_shape=jax.ShapeDtypeStruct((NG, 1), jnp.float32),
        scratch_shapes=[
            pltpu.VMEM((NG, D), jnp.float32),
            pltpu.VMEM((NG, D), jnp.float32),
        ],
    )(h, bi3, w_post, b_post, w_out, b_out)


def kernel(x, edge_index, edge_attr, batch_idx,
           W_emb, b_emb,
           W0, b0, gamma0, beta0, mmean0, mvar0,
           W1, b1, gamma1, beta1, mmean1, mvar1,
           W2, b2, gamma2, beta2, mmean2, mvar2,
           W_post, b_post, W_out, b_out):
    E = edge_index.shape[1]
    row = edge_index[0]
    col = edge_index[1]

    col3 = jnp.concatenate(
        [col, jnp.zeros((EP - E,), jnp.int32)]).reshape(NW, CPW, CHUNK)
    row3 = jnp.concatenate(
        [row, jnp.full((EP - E,), DUMMY, jnp.int32)]).reshape(NW, CPW, CHUNK)
    idx4 = jnp.stack([col3, row3], axis=2)
    ea_wide = jnp.concatenate(
        [_q(edge_attr), jnp.ones((E, 1), jnp.float32),
         jnp.zeros((E, D - DE - 1), jnp.float32)], axis=1)
    ea4 = jnp.concatenate(
        [ea_wide, jnp.zeros((EP - E, D), jnp.float32)]
    ).reshape(NW, CPW, CHUNK, D)
    x_p = jnp.concatenate([x, jnp.zeros((NP - N, D), jnp.float32)])
    bi3 = jnp.concatenate(
        [batch_idx, jnp.full((NP - N,), NG, jnp.int32)]).reshape(GRID, 1, NB)

    zeros_d = jnp.zeros((RPS, D), jnp.float32)

    st2 = _sc_edge_stats(ea4, idx4, zeros_d)
    h, h_q = _embed(_q(x_p), _q(W_emb), b_emb.reshape(1, D))

    for (W, b, g, be, mm, mv) in ((W0, b0, gamma0, beta0, mmean0, mvar0),
                                  (W1, b1, gamma1, beta1, mmean1, mvar1),
                                  (W2, b2, gamma2, beta2, mmean2, mvar2)):
        scale = (g / jnp.sqrt(mv + EPS)).reshape(1, D)
        shift = ((b - mm) * scale[0] + be).reshape(1, D)
        s2 = _sc_neighbor_sum(h_q, idx4, zeros_d)
        h, h_q = _layer_update(s2, st2, h, _q(W[:D]), _q(W[D:]), scale,
                               shift)

    out = _pool_head(h, bi3, _q(W_post), b_post.reshape(1, D),
                     _q(W_out), b_out.reshape(1, 1))
    return out[:, 0]
